# Initial kernel scaffold; baseline (speedup 1.0000x reference)
import jax
import jax.numpy as jnp
from jax.experimental import pallas as pl


def kernel(x, edge_index, W, b):
    raise NotImplementedError("write your pallas kernel here")


# speedup vs baseline: 15.4271x; 15.4271x over previous
import jax
import jax.numpy as jnp
from jax import lax
from jax.experimental import pallas as pl
from jax.experimental.pallas import tpu as pltpu
from jax.experimental.pallas import tpu_sc as plsc

N = 10000
E = 160000
D_IN = 256
H = 128
NC = 2
NS = 16
RPT = 632
RPT_LAST = N - 15 * RPT


def _tile_slab_copy(s, src_fn, dst_fn):
    off = pl.multiple_of(s * RPT, 8)

    @pl.when(s < NS - 1)
    def _():
        pltpu.sync_copy(src_fn(off, RPT), dst_fn(off, RPT))

    @pl.when(s == NS - 1)
    def _():
        pltpu.sync_copy(src_fn(off, RPT_LAST), dst_fn(off, RPT_LAST))


A_CH = 125
A_ROWS = E // (NC * NS * A_CH)
A_W = 128


def _deg_body(zero_ref, dst_ref, out_ref, deg_sp, idx_v, ones_v):
    c = lax.axis_index("c")
    s = lax.axis_index("s")

    def ofill(i, carry):
        for j in range(A_W // 16):
            ones_v[i, pl.ds(j * 16, 16)] = jnp.ones((16,), jnp.float32)
        return carry

    lax.fori_loop(0, A_CH, ofill, 0)

    _tile_slab_copy(
        s,
        lambda o, n: zero_ref.at[pl.ds(o, n)],
        lambda o, n: deg_sp.at[pl.ds(o, n)],
    )

    wid = c * NS + s
    pltpu.sync_copy(dst_ref.at[pl.ds(pl.multiple_of(wid * A_ROWS, 8), A_ROWS)],
                    idx_v)
    plsc.subcore_barrier()

    def scat(g, carry):
        pltpu.sync_copy(ones_v, deg_sp.at[idx_v.at[g]], add=True)
        return carry

    lax.fori_loop(0, A_ROWS, scat, 0)

    plsc.subcore_barrier()
    _tile_slab_copy(
        s,
        lambda o, n: deg_sp.at[pl.ds(o, n)],
        lambda o, n: out_ref.at[pl.ds(pl.multiple_of(c * N, 8) + o, n)],
    )


_deg_kernel = pl.kernel(
    _deg_body,
    out_type=jax.ShapeDtypeStruct((2 * N, A_W), jnp.float32),
    mesh=plsc.VectorSubcoreMesh(core_axis_name="c", subcore_axis_name="s"),
    scratch_types=[
        pltpu.VMEM_SHARED((N, A_W), jnp.float32),
        pltpu.VMEM((A_ROWS, A_CH), jnp.int32),
        pltpu.VMEM((A_CH, A_W), jnp.float32),
    ],
)

C_CH = 125
C_ROWS = E // (NS * C_CH)


def _agg_body(y_ref, srcb_ref, dst_ref, out_ref, acc_sp, sidx_v, didx_v,
              rows_v, sem):
    c = lax.axis_index("c")
    s = lax.axis_index("s")

    _tile_slab_copy(
        s,
        lambda o, n: y_ref.at[pl.ds(pl.multiple_of(c * N, 8) + o, n)],
        lambda o, n: acc_sp.at[pl.ds(o, n)],
    )

    row0 = pl.multiple_of(c * (E // C_CH) + s * C_ROWS, 8)
    pltpu.sync_copy(srcb_ref.at[pl.ds(row0, C_ROWS)], sidx_v)
    pltpu.sync_copy(dst_ref.at[pl.ds(pl.multiple_of(s * C_ROWS, 8), C_ROWS)],
                    didx_v)
    plsc.subcore_barrier()

    def step(g, carry):
        pltpu.async_copy(y_ref.at[sidx_v.at[g]], rows_v, sem).wait()
        pltpu.sync_copy(rows_v, acc_sp.at[didx_v.at[g]], add=True)
        return carry

    lax.fori_loop(0, C_ROWS, step, 0)

    plsc.subcore_barrier()
    _tile_slab_copy(
        s,
        lambda o, n: acc_sp.at[pl.ds(o, n)],
        lambda o, n: out_ref.at[pl.ds(pl.multiple_of(c * N, 8) + o, n)],
    )


_agg_kernel = pl.kernel(
    _agg_body,
    out_type=jax.ShapeDtypeStruct((2 * N, H), jnp.float32),
    mesh=plsc.VectorSubcoreMesh(core_axis_name="c", subcore_axis_name="s"),
    scratch_types=[
        pltpu.VMEM_SHARED((N, H), jnp.float32),
        pltpu.VMEM((C_ROWS, C_CH), jnp.int32),
        pltpu.VMEM((C_ROWS, C_CH), jnp.int32),
        pltpu.VMEM((C_CH, H), jnp.float32),
        pltpu.SemaphoreType.DMA,
    ],
)

R_B = 1000


def _mm_body(x_ref, w_ref, d0_ref, d1_ref, y_ref, dinv_ref):
    deg = d0_ref[:, 0] + d1_ref[:, 0] + 1.0
    dinv = lax.rsqrt(deg)
    xw = jnp.dot(x_ref[...], w_ref[...], preferred_element_type=jnp.float32)
    y = xw * dinv[:, None]
    y_ref[0] = y[:, :H]
    y_ref[1] = y[:, H:]
    dinv_ref[...] = dinv[:, None]


def _matmul_scale(x, w, degp):
    grid = N // R_B
    return pl.pallas_call(
        _mm_body,
        grid=(grid,),
        in_specs=[
            pl.BlockSpec((R_B, D_IN), lambda i: (i, 0)),
            pl.BlockSpec((D_IN, D_IN), lambda i: (0, 0)),
            pl.BlockSpec((R_B, A_W), lambda i: (i, 0)),
            pl.BlockSpec((R_B, A_W), lambda i: (i + grid, 0)),
        ],
        out_specs=[
            pl.BlockSpec((2, R_B, H), lambda i: (0, i, 0)),
            pl.BlockSpec((R_B, 1), lambda i: (i, 0)),
        ],
        out_shape=[
            jax.ShapeDtypeStruct((2, N, H), jnp.float32),
            jax.ShapeDtypeStruct((N, 1), jnp.float32),
        ],
    )(x, w, degp, degp)


def _fin_body(s_ref, dinv_ref, b_ref, o_ref):
    row = jnp.concatenate([s_ref[0], s_ref[1]], axis=1)
    o_ref[...] = jnp.maximum(row * dinv_ref[...] + b_ref[...], 0.0)


def _finalize(s2, dinv, b2):
    return pl.pallas_call(
        _fin_body,
        grid=(N // R_B,),
        in_specs=[
            pl.BlockSpec((2, R_B, H), lambda i: (0, i, 0)),
            pl.BlockSpec((R_B, 1), lambda i: (i, 0)),
            pl.BlockSpec((1, D_IN), lambda i: (0, 0)),
        ],
        out_specs=pl.BlockSpec((R_B, D_IN), lambda i: (i, 0)),
        out_shape=jax.ShapeDtypeStruct((N, D_IN), jnp.float32),
    )(s2, dinv, b2)


def kernel(x, edge_index, W, b):
    src = edge_index[0].astype(jnp.int32)
    dst = edge_index[1].astype(jnp.int32)

    dstA = dst.reshape(E // A_CH, A_CH)
    dstC = dst.reshape(E // C_CH, C_CH)
    srcb = jnp.concatenate([src, src + N]).reshape(2 * (E // C_CH), C_CH)

    degp = _deg_kernel(jnp.zeros((N, A_W), jnp.float32), dstA)
    y2, dinv = _matmul_scale(x, W, degp)
    y_flat = y2.reshape(2 * N, H)
    s_flat = _agg_kernel(y_flat, srcb, dstC)
    return _finalize(s_flat.reshape(2, N, H), dinv, b.reshape(1, D_IN))


# speedup vs baseline: 20.3225x; 1.3173x over previous
import jax
import jax.numpy as jnp
from jax import lax
from jax.experimental import pallas as pl
from jax.experimental.pallas import tpu as pltpu
from jax.experimental.pallas import tpu_sc as plsc

N = 10000
E = 160000
D_IN = 256
H = 128
NC = 2
NS = 16
RPT = 632
RPT_LAST = N - 15 * RPT


def _tile_slab_copy(s, src_fn, dst_fn):
    off = pl.multiple_of(s * RPT, 8)

    @pl.when(s < NS - 1)
    def _():
        pltpu.sync_copy(src_fn(off, RPT), dst_fn(off, RPT))

    @pl.when(s == NS - 1)
    def _():
        pltpu.sync_copy(src_fn(off, RPT_LAST), dst_fn(off, RPT_LAST))


A_CH = 125
A_ROWS = E // (NC * NS * A_CH)
A_W = 128


def _deg_body(zero_ref, dst_ref, out_ref, deg_sp, idx_v, ones_v):
    c = lax.axis_index("c")
    s = lax.axis_index("s")

    def ofill(i, carry):
        for j in range(A_W // 16):
            ones_v[i, pl.ds(j * 16, 16)] = jnp.ones((16,), jnp.float32)
        return carry

    lax.fori_loop(0, A_CH, ofill, 0)

    _tile_slab_copy(
        s,
        lambda o, n: zero_ref.at[pl.ds(o, n)],
        lambda o, n: deg_sp.at[pl.ds(o, n)],
    )

    wid = c * NS + s
    pltpu.sync_copy(dst_ref.at[pl.ds(pl.multiple_of(wid * A_ROWS, 8), A_ROWS)],
                    idx_v)
    plsc.subcore_barrier()

    def scat(g, carry):
        pltpu.sync_copy(ones_v, deg_sp.at[idx_v.at[g]], add=True)
        return carry

    lax.fori_loop(0, A_ROWS, scat, 0)

    plsc.subcore_barrier()
    _tile_slab_copy(
        s,
        lambda o, n: deg_sp.at[pl.ds(o, n)],
        lambda o, n: out_ref.at[pl.ds(pl.multiple_of(c * N, 8) + o, n)],
    )


_deg_kernel = pl.kernel(
    _deg_body,
    out_type=jax.ShapeDtypeStruct((2 * N, A_W), jnp.float32),
    mesh=plsc.VectorSubcoreMesh(core_axis_name="c", subcore_axis_name="s"),
    scratch_types=[
        pltpu.VMEM_SHARED((N, A_W), jnp.float32),
        pltpu.VMEM((A_ROWS, A_CH), jnp.int32),
        pltpu.VMEM((A_CH, A_W), jnp.float32),
    ],
)

C_CH = 125
C_ROWS = E // (NS * C_CH)
C_HALF = C_ROWS // 2


def _agg_body(y_ref, srcb_ref, dst_ref, out_ref, acc_sp, sidx_v, didx_v,
              rows0_v, rows1_v, sem0, sem1):
    c = lax.axis_index("c")
    s = lax.axis_index("s")

    _tile_slab_copy(
        s,
        lambda o, n: y_ref.at[pl.ds(pl.multiple_of(c * N, 8) + o, n)],
        lambda o, n: acc_sp.at[pl.ds(o, n)],
    )

    row0 = pl.multiple_of(c * (E // C_CH) + s * C_ROWS, 8)
    dbase = pl.multiple_of(s * C_ROWS, 8)
    plsc.subcore_barrier()

    for hb in (0, C_HALF):
        pltpu.sync_copy(srcb_ref.at[pl.ds(row0 + hb, C_HALF)], sidx_v)
        pltpu.sync_copy(dst_ref.at[pl.ds(dbase + hb, C_HALF)], didx_v)
        pltpu.async_copy(y_ref.at[sidx_v.at[0]], rows0_v, sem0)

        def step(h, carry):
            g0 = 2 * h
            pltpu.async_copy(y_ref.at[sidx_v.at[g0 + 1]], rows1_v, sem1)
            pltpu.make_async_copy(y_ref.at[sidx_v.at[g0]], rows0_v,
                                  sem0).wait()
            pltpu.sync_copy(rows0_v, acc_sp.at[didx_v.at[g0]], add=True)

            @pl.when(g0 + 2 < C_HALF)
            def _():
                pltpu.async_copy(y_ref.at[sidx_v.at[g0 + 2]], rows0_v, sem0)

            pltpu.make_async_copy(y_ref.at[sidx_v.at[g0 + 1]], rows1_v,
                                  sem1).wait()
            pltpu.sync_copy(rows1_v, acc_sp.at[didx_v.at[g0 + 1]], add=True)
            return carry

        lax.fori_loop(0, C_HALF // 2, step, 0)

    plsc.subcore_barrier()
    _tile_slab_copy(
        s,
        lambda o, n: acc_sp.at[pl.ds(o, n)],
        lambda o, n: out_ref.at[pl.ds(pl.multiple_of(c * N, 8) + o, n)],
    )


_agg_kernel = pl.kernel(
    _agg_body,
    out_type=jax.ShapeDtypeStruct((2 * N, H), jnp.float32),
    mesh=plsc.VectorSubcoreMesh(core_axis_name="c", subcore_axis_name="s"),
    scratch_types=[
        pltpu.VMEM_SHARED((N, H), jnp.float32),
        pltpu.VMEM((C_HALF, C_CH), jnp.int32),
        pltpu.VMEM((C_HALF, C_CH), jnp.int32),
        pltpu.VMEM((C_CH, H), jnp.float32),
        pltpu.VMEM((C_CH, H), jnp.float32),
        pltpu.SemaphoreType.DMA,
        pltpu.SemaphoreType.DMA,
    ],
)

R_B = 1000


def _mm_body(x_ref, w_ref, d0_ref, d1_ref, y_ref, dinv_ref):
    deg = d0_ref[:, 0] + d1_ref[:, 0] + 1.0
    dinv = lax.rsqrt(deg)
    xw = jnp.dot(x_ref[...], w_ref[...], preferred_element_type=jnp.float32)
    y = xw * dinv[:, None]
    y_ref[0] = y[:, :H]
    y_ref[1] = y[:, H:]
    dinv_ref[...] = dinv[:, None]


def _matmul_scale(x, w, degp):
    grid = N // R_B
    return pl.pallas_call(
        _mm_body,
        grid=(grid,),
        in_specs=[
            pl.BlockSpec((R_B, D_IN), lambda i: (i, 0)),
            pl.BlockSpec((D_IN, D_IN), lambda i: (0, 0)),
            pl.BlockSpec((R_B, A_W), lambda i: (i, 0)),
            pl.BlockSpec((R_B, A_W), lambda i: (i + grid, 0)),
        ],
        out_specs=[
            pl.BlockSpec((2, R_B, H), lambda i: (0, i, 0)),
            pl.BlockSpec((R_B, 1), lambda i: (i, 0)),
        ],
        out_shape=[
            jax.ShapeDtypeStruct((2, N, H), jnp.float32),
            jax.ShapeDtypeStruct((N, 1), jnp.float32),
        ],
    )(x, w, degp, degp)


def _fin_body(s_ref, dinv_ref, b_ref, o_ref):
    row = jnp.concatenate([s_ref[0], s_ref[1]], axis=1)
    o_ref[...] = jnp.maximum(row * dinv_ref[...] + b_ref[...], 0.0)


def _finalize(s2, dinv, b2):
    return pl.pallas_call(
        _fin_body,
        grid=(N // R_B,),
        in_specs=[
            pl.BlockSpec((2, R_B, H), lambda i: (0, i, 0)),
            pl.BlockSpec((R_B, 1), lambda i: (i, 0)),
            pl.BlockSpec((1, D_IN), lambda i: (0, 0)),
        ],
        out_specs=pl.BlockSpec((R_B, D_IN), lambda i: (i, 0)),
        out_shape=jax.ShapeDtypeStruct((N, D_IN), jnp.float32),
    )(s2, dinv, b2)


def kernel(x, edge_index, W, b):
    src = edge_index[0].astype(jnp.int32)
    dst = edge_index[1].astype(jnp.int32)

    dstA = dst.reshape(E // A_CH, A_CH)
    dstC = dst.reshape(E // C_CH, C_CH)
    srcb = jnp.concatenate([src, src + N]).reshape(2 * (E // C_CH), C_CH)

    degp = _deg_kernel(jnp.zeros((N, A_W), jnp.float32), dstA)
    y2, dinv = _matmul_scale(x, W, degp)
    y_flat = y2.reshape(2 * N, H)
    s_flat = _agg_kernel(y_flat, srcb, dstC)
    return _finalize(s_flat.reshape(2, N, H), dinv, b.reshape(1, D_IN))
